# baseline (speedup 1.0000x reference)
import functools

import jax
import jax.numpy as jnp
import numpy as np
from jax import lax
from jax.experimental import pallas as pl
from jax.experimental.pallas import tpu as pltpu
from jax.experimental.pallas import tpu_sc as plsc

_NUM_BUCKETS = 32
_MAX_DISTANCE = 128
_NUM_HEADS = 16
_S = 2048
_T = 2 * _S - 1
_TP = 4096
_B = 16
_NBLK = _S // 2 // _B


def _diag_bucket_table() -> np.ndarray:
    d = np.arange(-(_S - 1), _S, dtype=np.int32)
    neg = -d
    nb = _NUM_BUCKETS // 2
    me = nb // 2
    b = (neg < 0).astype(np.int32) * nb
    neg = np.abs(neg)
    large = me + (
        np.log(neg.astype(np.float32) / me + np.finfo(np.float32).eps)
        / np.log(_MAX_DISTANCE / me)
        * (nb - me)
    ).astype(np.int32)
    large = np.minimum(large, nb - 1)
    b = b + np.where(neg < me, neg, large)
    return np.concatenate([b, b[-1:]]).astype(np.int32)


_BUCKET_TABLE = _diag_bucket_table()


def _rpb_body(bt_hbm, e_hbm, out_hbm, bt_v, e_v, d_v, dmat_v, sem):
    nc = 2
    wid = lax.axis_index("s") * nc + lax.axis_index("c")
    h = wid % _NUM_HEADS
    row_base = (wid // _NUM_HEADS) * (_S // 2)

    pltpu.sync_copy(bt_hbm, bt_v)
    pltpu.sync_copy(e_hbm, e_v)

    iota = lax.iota(jnp.int32, 16)
    h_vec = jnp.full((16,), h, dtype=jnp.int32)

    def dbody(k, c):
        bidx = bt_v[pl.ds(k * 16, 16)]
        d_v[pl.ds(k * 16, 16)] = plsc.load_gather(e_v, [h_vec, bidx])
        return c

    lax.fori_loop(0, _TP // 16, dbody, 0)

    def mbody(n, c):
        b = n // (_TP // 16)
        k = n % (_TP // 16)
        idx = jnp.maximum(k * 16 + iota - b - 1, 0)
        dmat_v[b, pl.ds(k * 16, 16)] = plsc.load_gather(d_v, [idx])
        return c

    lax.fori_loop(0, _B * (_TP // 16), mbody, 0)

    def _blk_copy(blk):
        i0 = row_base + blk * _B
        off = _S - i0
        return pltpu.make_async_copy(
            dmat_v.at[:, pl.ds(off, _S)],
            out_hbm.at[pl.ds(h * _S + i0, _B)],
            sem,
        )

    def fire(blk, c):
        _blk_copy(blk).start()
        return c

    lax.fori_loop(0, _NBLK, fire, 0)

    def drain(blk, c):
        _blk_copy(blk).wait()
        return c

    lax.fori_loop(0, _NBLK, drain, 0)


@jax.jit
def _rpb(rel_embedding):
    e = jnp.asarray(rel_embedding, jnp.float32)
    bt = jnp.asarray(_BUCKET_TABLE)
    call = pl.kernel(
        _rpb_body,
        out_type=jax.ShapeDtypeStruct((_NUM_HEADS * _S, _S), jnp.float32),
        mesh=plsc.VectorSubcoreMesh(core_axis_name="c", subcore_axis_name="s"),
        scratch_types=[
            pltpu.VMEM((_TP,), jnp.int32),
            pltpu.VMEM((_NUM_HEADS, _NUM_BUCKETS), jnp.float32),
            pltpu.VMEM((_TP,), jnp.float32),
            pltpu.VMEM((_B, _TP), jnp.float32),
            pltpu.SemaphoreType.DMA,
        ],
        compiler_params=pltpu.CompilerParams(
            use_tc_tiling_on_sc=False, needs_layout_passes=False
        ),
    )
    out = call(bt, e)
    return out.reshape(1, _NUM_HEADS, _S, _S)


def kernel(rel_embedding, q_seqlen, k_seqlen):
    del q_seqlen, k_seqlen
    return _rpb(rel_embedding)


# speedup vs baseline: 1.3805x; 1.3805x over previous
import functools

import jax
import jax.numpy as jnp
import numpy as np
from jax import lax
from jax.experimental import pallas as pl
from jax.experimental.pallas import tpu as pltpu
from jax.experimental.pallas import tpu_sc as plsc

_NUM_BUCKETS = 32
_MAX_DISTANCE = 128
_NUM_HEADS = 16
_S = 2048
_TP = 4096
_BR = 128
_NC = 2
_PAD = 128
_HTC = 10
_HSC = _NUM_HEADS - _HTC
_HPC = _HSC // _NC
_TI = 512


def _diag_bucket_table() -> np.ndarray:
    d = np.arange(-(_S - 1), _S, dtype=np.int32)
    neg = -d
    nb = _NUM_BUCKETS // 2
    me = nb // 2
    b = (neg < 0).astype(np.int32) * nb
    neg = np.abs(neg)
    large = me + (
        np.log(neg.astype(np.float32) / me + np.finfo(np.float32).eps)
        / np.log(_MAX_DISTANCE / me)
        * (nb - me)
    ).astype(np.int32)
    large = np.minimum(large, nb - 1)
    b = b + np.where(neg < me, neg, large)
    return np.concatenate([b, b[-1:]]).astype(np.int32)


_BUCKET_TABLE = _diag_bucket_table()
_ONE_HOT = (_BUCKET_TABLE[None, :] == np.arange(_NUM_BUCKETS)[:, None]).astype(
    np.float32
)


def _sc_body(bt_hbm, e_hbm, out_hbm, bt_v, e_v, d_v, bld_v, dmat_s, sem):
    core = lax.axis_index("c")
    tid = lax.axis_index("s")

    pltpu.sync_copy(bt_hbm, bt_v)
    pltpu.sync_copy(e_hbm, e_v)

    def out_dma(hh, buf):
        sh = core * _HPC + hh
        i0 = _BR * tid
        return pltpu.make_async_copy(
            dmat_s.at[buf, :, pl.ds(_S - i0, _S)],
            out_hbm.at[pl.ds(sh * _S + i0, _BR)],
            sem,
        )

    for hh in range(_HPC):
        buf = hh % 2
        h = _HTC + core * _HPC + hh
        h_vec = jnp.full((16,), h, dtype=jnp.int32)

        @plsc.parallel_loop(0, _TP // 16, unroll=4)
        def dbody(k, h_vec=h_vec):
            bidx = bt_v[pl.ds(k * 16, 16)]
            d_v[pl.ds(_PAD + k * 16, 16)] = plsc.load_gather(e_v, [h_vec, bidx])

        for r in range(8):
            sh = _PAD - (8 * tid + r + 1)

            @plsc.parallel_loop(8, _TP // 16, unroll=4)
            def rbody(k, sh=sh, r=r):
                bld_v[r, pl.ds(k * 16, 16)] = d_v[pl.ds(k * 16 + sh, 16)]

        if hh >= 2:
            out_dma(hh - 2, buf).wait()
            plsc.subcore_barrier()

        pltpu.sync_copy(bld_v, dmat_s.at[buf, pl.ds(8 * tid, 8), :])
        plsc.subcore_barrier()
        out_dma(hh, buf).start()

    for hh in range(max(0, _HPC - 2), _HPC):
        out_dma(hh, hh % 2).wait()


def _tc_body(e_ref, oh_ref, out_ref):
    h = pl.program_id(0)
    e_row = e_ref[pl.ds(h, 1), :]
    d = jnp.dot(e_row, oh_ref[...], preferred_element_type=jnp.float32)
    db = jnp.broadcast_to(d, (_TI, _TP))
    i0 = pl.program_id(1) * _TI
    rolled = pltpu.roll(db, i0 + _TP - (_S - 1), axis=1)
    rolled = pltpu.roll(rolled, 0, axis=1, stride=1, stride_axis=0)
    out_ref[...] = rolled[:, :_S]


@jax.jit
def _rpb(rel_embedding):
    e = jnp.asarray(rel_embedding, jnp.float32)
    bt = jnp.asarray(_BUCKET_TABLE)
    oh = jnp.asarray(_ONE_HOT)

    sc_call = pl.kernel(
        _sc_body,
        out_type=jax.ShapeDtypeStruct((_HSC * _S, _S), jnp.float32),
        mesh=plsc.VectorSubcoreMesh(core_axis_name="c", subcore_axis_name="s"),
        scratch_types=[
            pltpu.VMEM((_TP,), jnp.int32),
            pltpu.VMEM((_NUM_HEADS, _NUM_BUCKETS), jnp.float32),
            pltpu.VMEM((_PAD + _TP,), jnp.float32),
            pltpu.VMEM((8, _TP), jnp.float32),
            pltpu.VMEM_SHARED((2, _BR, _TP), jnp.float32),
            pltpu.SemaphoreType.DMA,
        ],
        compiler_params=pltpu.CompilerParams(needs_layout_passes=False),
    )
    sc_out = sc_call(bt, e)

    tc_out = pl.pallas_call(
        _tc_body,
        out_shape=jax.ShapeDtypeStruct((_HTC * _S, _S), jnp.float32),
        grid=(_HTC, _S // _TI),
        in_specs=[
            pl.BlockSpec((_NUM_HEADS, _NUM_BUCKETS), lambda h, r: (0, 0)),
            pl.BlockSpec((_NUM_BUCKETS, _TP), lambda h, r: (0, 0)),
        ],
        out_specs=pl.BlockSpec(
            (_TI, _S), lambda h, r: (h * (_S // _TI) + r, 0)
        ),
    )(e, oh)

    out = jnp.concatenate([tc_out, sc_out], axis=0)
    return out.reshape(1, _NUM_HEADS, _S, _S)


def kernel(rel_embedding, q_seqlen, k_seqlen):
    del q_seqlen, k_seqlen
    return _rpb(rel_embedding)


# speedup vs baseline: 3.1023x; 2.2473x over previous
import jax
import jax.numpy as jnp
import numpy as np
from jax import lax
from jax.experimental import pallas as pl
from jax.experimental.pallas import tpu as pltpu
from jax.experimental.pallas import tpu_sc as plsc

_NUM_BUCKETS = 32
_MAX_DISTANCE = 128
_NUM_HEADS = 16
_S = 2048
_TP = 4096
_BR = 128
_NC = 2
_PAD = 128
_HSC = 4
_HTC = _NUM_HEADS - _HSC
_HPC = _HSC // _NC
_TI = 512


def _diag_bucket_table() -> np.ndarray:
    d = np.arange(-(_S - 1), _S, dtype=np.int32)
    neg = -d
    nb = _NUM_BUCKETS // 2
    me = nb // 2
    b = (neg < 0).astype(np.int32) * nb
    neg = np.abs(neg)
    large = me + (
        np.log(neg.astype(np.float32) / me + np.finfo(np.float32).eps)
        / np.log(_MAX_DISTANCE / me)
        * (nb - me)
    ).astype(np.int32)
    large = np.minimum(large, nb - 1)
    b = b + np.where(neg < me, neg, large)
    return np.concatenate([b, b[-1:]]).astype(np.int32)


_BUCKET_TABLE = _diag_bucket_table()
_ONE_HOT = (_BUCKET_TABLE[None, :] == np.arange(_NUM_BUCKETS)[:, None]).astype(
    np.float32
)


def _sc_body(bt_hbm, e_hbm, out_hbm, bt_v, e_v, d_v, bld_v, dmat_s, sem):
    core = lax.axis_index("c")
    tid = lax.axis_index("s")

    pltpu.sync_copy(bt_hbm, bt_v)
    pltpu.sync_copy(e_hbm, e_v)

    def out_dma(hh, buf):
        sh = core * _HPC + hh
        i0 = _BR * tid
        return pltpu.make_async_copy(
            dmat_s.at[buf, :, pl.ds(_S - i0, _S)],
            out_hbm.at[pl.ds(sh * _S + i0, _BR)],
            sem,
        )

    for hh in range(_HPC):
        buf = hh % 2
        h = core * _HPC + hh
        h_vec = jnp.full((16,), h, dtype=jnp.int32)

        @plsc.parallel_loop(0, _TP // 16, unroll=4)
        def dbody(k, h_vec=h_vec):
            bidx = bt_v[pl.ds(k * 16, 16)]
            d_v[pl.ds(_PAD + k * 16, 16)] = plsc.load_gather(e_v, [h_vec, bidx])

        for r in range(8):
            sh = _PAD - (8 * tid + r + 1)

            @plsc.parallel_loop(8, _TP // 16, unroll=4)
            def rbody(k, sh=sh, r=r):
                bld_v[r, pl.ds(k * 16, 16)] = d_v[pl.ds(k * 16 + sh, 16)]

        if hh >= 2:
            out_dma(hh - 2, buf).wait()
            plsc.subcore_barrier()

        pltpu.sync_copy(bld_v, dmat_s.at[buf, pl.ds(8 * tid, 8), :])
        plsc.subcore_barrier()
        out_dma(hh, buf).start()

    for hh in range(max(0, _HPC - 2), _HPC):
        out_dma(hh, hh % 2).wait()


def _tc_body(e_ref, oh_ref, big_ref, out_ref):
    del big_ref
    h = pl.program_id(0) + _HSC
    e_row = e_ref[pl.ds(h, 1), :]
    d = jnp.dot(e_row, oh_ref[...], preferred_element_type=jnp.float32)
    i0 = pl.program_id(1) * _TI
    d = pltpu.roll(d, i0 + _TP - (_S - 1), axis=1)
    db = jnp.broadcast_to(d, (_TI, _TP))
    rolled = pltpu.roll(db, 0, axis=1, stride=1, stride_axis=0)
    out_ref[...] = rolled[:, :_S]


@jax.jit
def _rpb(rel_embedding):
    e = jnp.asarray(rel_embedding, jnp.float32)
    bt = jnp.asarray(_BUCKET_TABLE)
    oh = jnp.asarray(_ONE_HOT)

    sc_call = pl.kernel(
        _sc_body,
        out_type=jax.ShapeDtypeStruct((_NUM_HEADS * _S, _S), jnp.float32),
        mesh=plsc.VectorSubcoreMesh(core_axis_name="c", subcore_axis_name="s"),
        scratch_types=[
            pltpu.VMEM((_TP,), jnp.int32),
            pltpu.VMEM((_NUM_HEADS, _NUM_BUCKETS), jnp.float32),
            pltpu.VMEM((_PAD + _TP,), jnp.float32),
            pltpu.VMEM((8, _TP), jnp.float32),
            pltpu.VMEM_SHARED((2, _BR, _TP), jnp.float32),
            pltpu.SemaphoreType.DMA,
        ],
        compiler_params=pltpu.CompilerParams(needs_layout_passes=False),
    )
    sc_out = sc_call(bt, e)

    out = pl.pallas_call(
        _tc_body,
        out_shape=jax.ShapeDtypeStruct((_NUM_HEADS * _S, _S), jnp.float32),
        grid=(_HTC, _S // _TI),
        in_specs=[
            pl.BlockSpec((_NUM_HEADS, _NUM_BUCKETS), lambda h, r: (0, 0)),
            pl.BlockSpec((_NUM_BUCKETS, _TP), lambda h, r: (0, 0)),
            pl.BlockSpec(memory_space=pl.ANY),
        ],
        out_specs=pl.BlockSpec(
            (_TI, _S), lambda h, r: ((_HSC + h) * (_S // _TI) + r, 0)
        ),
        input_output_aliases={2: 0},
    )(e, oh, sc_out)

    return out.reshape(1, _NUM_HEADS, _S, _S)


def kernel(rel_embedding, q_seqlen, k_seqlen):
    del q_seqlen, k_seqlen
    return _rpb(rel_embedding)
